# baseline (speedup 1.0000x reference)
import functools

import jax
import jax.numpy as jnp
from jax.experimental import pallas as pl
from jax.experimental.pallas import tpu as pltpu

_EPS = 1e-5
_NEG = -9e15
_LOG2E = 1.4426950408889634


def _fused_kernel(nsteps, br, x_ref, xblk_ref, adj_ref, watt_ref, a1s_ref,
                  a2s_ref, wgc_ref, gamma_ref, beta_ref, rec_ref,
                  gext_ref, e2row_ref, sup_ref, out_ref, outn_ref,
                  adjbf_ref):
    p = pl.program_id(0)
    i = pl.program_id(1)
    n, hd2 = x_ref.shape
    hd1 = wgc_ref.shape[1]

    @pl.when(p == 0)
    def _attention_phase():
        @pl.when(i == 0)
        def _init():
            wh = jnp.dot(x_ref[...], watt_ref[...])
            gext_ref[:, :hd1] = jnp.dot(wh, wgc_ref[...])
            gext_ref[:, hd1:hd1 + 1] = jnp.ones((n, 1), jnp.float32)
            e2row_ref[...] = jax.lax.dot_general(
                a2s_ref[...], wh, (((1,), (1,)), ((), ())))

        whb = jnp.dot(xblk_ref[...], watt_ref[...])
        e1b = jax.lax.dot_general(
            whb, a1s_ref[...], (((1,), (1,)), ((), ())))
        e = e1b + e2row_ref[...]
        e = jnp.maximum(e, 0.2 * e)
        adjb = adj_ref[...]
        adjbf_ref[pl.ds(i * br, br), :] = adjb.astype(jnp.bfloat16)
        m = jnp.where(adjb > 0, e, _NEG)
        mmax = jnp.max(m, axis=1, keepdims=True)
        pexp = jnp.exp2(m - mmax)
        res = jnp.dot(pexp, gext_ref[...])
        sup_ref[pl.ds(i * br, br), :] = (
            res[:, :hd1] / res[:, hd1:hd1 + 1])

    @pl.when(p == 1)
    def _aggregate_phase():
        adjb = adjbf_ref[pl.ds(i * br, br), :]
        supb = sup_ref[...].astype(jnp.bfloat16)
        out_ref[pl.ds(i * br, br), :] = jnp.maximum(
            jnp.dot(adjb, supb, preferred_element_type=jnp.float32), 0.0)

    @pl.when(p == 2)
    def _decode_phase():
        @pl.when(i == 0)
        def _normalize():
            o = out_ref[...]
            mean = jnp.mean(o, axis=0, keepdims=True)
            cen = o - mean
            var = jnp.mean(cen * cen, axis=0, keepdims=True)
            scale = jax.lax.rsqrt(var + _EPS) * gamma_ref[...]
            outn_ref[...] = cen * scale + beta_ref[...]

        blk = outn_ref[pl.ds(i * br, br), :]
        rec_ref[...] = jax.lax.dot_general(
            blk, outn_ref[...], (((1,), (1,)), ((), ())))


def kernel(encoder_layer_2, adj, W_att, a_att, W_gc, bn_gamma, bn_beta):
    n, hd2 = encoder_layer_2.shape
    hd1 = W_gc.shape[1]
    br = min(128, n)
    nsteps = n // br

    a1s = (a_att[:hd2] * _LOG2E).reshape(1, hd2)
    a2s = (a_att[hd2:] * _LOG2E).reshape(1, hd2)
    gamma = bn_gamma.reshape(1, hd1)
    beta = bn_beta.reshape(1, hd1)

    last = nsteps - 1
    rec = pl.pallas_call(
        functools.partial(_fused_kernel, nsteps, br),
        grid=(3, nsteps),
        in_specs=[
            pl.BlockSpec((n, hd2), lambda p, i: (0, 0)),
            pl.BlockSpec((br, hd2), lambda p, i: (jnp.where(p == 0, i, 0), 0)),
            pl.BlockSpec((br, n),
                         lambda p, i: (jnp.where(p == 0, i, last), 0)),
            pl.BlockSpec((hd2, hd2), lambda p, i: (0, 0)),
            pl.BlockSpec((1, hd2), lambda p, i: (0, 0)),
            pl.BlockSpec((1, hd2), lambda p, i: (0, 0)),
            pl.BlockSpec((hd2, hd1), lambda p, i: (0, 0)),
            pl.BlockSpec((1, hd1), lambda p, i: (0, 0)),
            pl.BlockSpec((1, hd1), lambda p, i: (0, 0)),
        ],
        out_specs=pl.BlockSpec((br, n), lambda p, i: (jnp.where(p == 2, i, 0), 0)),
        out_shape=jax.ShapeDtypeStruct((n, n), jnp.float32),
        scratch_shapes=[
            pltpu.VMEM((n, hd1 + 1), jnp.float32),
            pltpu.VMEM((1, n), jnp.float32),
            pltpu.VMEM((n, hd1), jnp.float32),
            pltpu.VMEM((n, hd1), jnp.float32),
            pltpu.VMEM((n, hd1), jnp.float32),
            pltpu.VMEM((n, n), jnp.bfloat16),
        ],
    )(encoder_layer_2, encoder_layer_2, adj, W_att, a1s, a2s, W_gc,
      gamma, beta)

    return rec


# speedup vs baseline: 1.2713x; 1.2713x over previous
import functools

import jax
import jax.numpy as jnp
from jax.experimental import pallas as pl
from jax.experimental.pallas import tpu as pltpu

_EPS = 1e-5
_NEG = -9e15
_LOG2E = 1.4426950408889634


def _fused_kernel(nsteps, br, x_ref, xblk_ref, adj_ref, watt_ref, a1s_ref,
                  a2s_ref, wgc_ref, gamma_ref, beta_ref, rec_ref,
                  gext_ref, e2row_ref, sup_ref, out_ref, outn_ref):
    p = pl.program_id(0)
    i = pl.program_id(1)
    n, hd2 = x_ref.shape
    hd1 = wgc_ref.shape[1]

    @pl.when(p == 0)
    def _attention_phase():
        @pl.when(i == 0)
        def _init():
            wh = jnp.dot(x_ref[...], watt_ref[...])
            gext_ref[:, :hd1] = jnp.dot(wh, wgc_ref[...])
            gext_ref[:, hd1:hd1 + 1] = jnp.ones((n, 1), jnp.float32)
            e2row_ref[...] = jax.lax.dot_general(
                a2s_ref[...], wh, (((1,), (1,)), ((), ())))

        whb = jnp.dot(xblk_ref[...], watt_ref[...])
        e1b = jax.lax.dot_general(
            whb, a1s_ref[...], (((1,), (1,)), ((), ())))
        e = e1b + e2row_ref[...]
        e = jnp.maximum(e, 0.2 * e)
        m = jnp.where(adj_ref[...] > 0, e, _NEG)
        mmax = jnp.max(m, axis=1, keepdims=True)
        pexp = jnp.exp2(m - mmax)
        res = jnp.dot(pexp, gext_ref[...])
        sup_ref[pl.ds(i * br, br), :] = (
            res[:, :hd1] / res[:, hd1:hd1 + 1])

    @pl.when(p == 1)
    def _aggregate_phase():
        out_ref[pl.ds(i * br, br), :] = jnp.maximum(
            jnp.dot(adj_ref[...], sup_ref[...]), 0.0)

    @pl.when(p == 2)
    def _decode_phase():
        @pl.when(i == 0)
        def _normalize():
            o = out_ref[...]
            mean = jnp.mean(o, axis=0, keepdims=True)
            cen = o - mean
            var = jnp.mean(cen * cen, axis=0, keepdims=True)
            scale = jax.lax.rsqrt(var + _EPS) * gamma_ref[...]
            outn_ref[...] = cen * scale + beta_ref[...]

        blk = outn_ref[pl.ds(i * br, br), :]
        rec_ref[...] = jax.lax.dot_general(
            blk, outn_ref[...], (((1,), (1,)), ((), ())))


def kernel(encoder_layer_2, adj, W_att, a_att, W_gc, bn_gamma, bn_beta):
    n, hd2 = encoder_layer_2.shape
    hd1 = W_gc.shape[1]
    br = min(512, n)
    nsteps = n // br

    a1s = (a_att[:hd2] * _LOG2E).reshape(1, hd2)
    a2s = (a_att[hd2:] * _LOG2E).reshape(1, hd2)
    gamma = bn_gamma.reshape(1, hd1)
    beta = bn_beta.reshape(1, hd1)

    last = nsteps - 1
    rec = pl.pallas_call(
        functools.partial(_fused_kernel, nsteps, br),
        grid=(3, nsteps),
        in_specs=[
            pl.BlockSpec((n, hd2), lambda p, i: (0, 0)),
            pl.BlockSpec((br, hd2), lambda p, i: (jnp.where(p == 0, i, 0), 0)),
            pl.BlockSpec((br, n),
                         lambda p, i: (jnp.where(p < 2, i, last), 0)),
            pl.BlockSpec((hd2, hd2), lambda p, i: (0, 0)),
            pl.BlockSpec((1, hd2), lambda p, i: (0, 0)),
            pl.BlockSpec((1, hd2), lambda p, i: (0, 0)),
            pl.BlockSpec((hd2, hd1), lambda p, i: (0, 0)),
            pl.BlockSpec((1, hd1), lambda p, i: (0, 0)),
            pl.BlockSpec((1, hd1), lambda p, i: (0, 0)),
        ],
        out_specs=pl.BlockSpec((br, n), lambda p, i: (jnp.where(p == 2, i, 0), 0)),
        out_shape=jax.ShapeDtypeStruct((n, n), jnp.float32),
        scratch_shapes=[
            pltpu.VMEM((n, hd1 + 1), jnp.float32),
            pltpu.VMEM((1, n), jnp.float32),
            pltpu.VMEM((n, hd1), jnp.float32),
            pltpu.VMEM((n, hd1), jnp.float32),
            pltpu.VMEM((n, hd1), jnp.float32),
        ],
    )(encoder_layer_2, encoder_layer_2, adj, W_att, a1s, a2s, W_gc,
      gamma, beta)

    return rec
